# baseline (speedup 1.0000x reference)
import functools

import jax
import jax.numpy as jnp
from jax import lax
from jax.experimental import pallas as pl
from jax.experimental.pallas import tpu as pltpu
from jax.experimental.pallas import tpu_sc as plsc

N = 10000
E = 320000
D = 128
C = 40
CP = 128

NC = 2
NS = 16
NT = NC * NS
BL = 128
NB = 80
NBC = 16
EPT = NB * BL
PADE = NT * EPT - E
NPAD = 10112
RPT = NPAD // NS

_mesh = plsc.VectorSubcoreMesh(
    core_axis_name="c", subcore_axis_name="s", num_cores=NC, num_subcores=NS)


@functools.partial(
    pl.kernel,
    out_type=jax.ShapeDtypeStruct((NC, NPAD, D), jnp.float32),
    mesh=_mesh,
    scratch_types=[
        pltpu.VMEM((NB, BL), jnp.int32),
        pltpu.VMEM((BL, D), jnp.float32),
        pltpu.VMEM_SHARED((NPAD, D), jnp.float32),
    ],
)
def _sc_hist(dst_hbm, zeros_hbm, ones_hbm, out_hbm, idx_v, ones_v, acc_sh):
    cid = lax.axis_index("c")
    sid = lax.axis_index("s")
    t = cid * NS + sid
    pltpu.sync_copy(dst_hbm.at[t], idx_v)
    pltpu.sync_copy(ones_hbm, ones_v)
    r0 = sid * RPT
    pltpu.sync_copy(zeros_hbm.at[pl.ds(r0, RPT)], acc_sh.at[pl.ds(r0, RPT)])
    plsc.subcore_barrier()

    def body(j, carry):
        pltpu.sync_copy(ones_v, acc_sh.at[idx_v.at[j]], add=True)
        return carry

    lax.fori_loop(0, NB, body, 0)
    plsc.subcore_barrier()
    pltpu.sync_copy(acc_sh.at[pl.ds(r0, RPT)], out_hbm.at[cid, pl.ds(r0, RPT)])


def _make_sc_conv(width):
    @functools.partial(
        pl.kernel,
        out_type=jax.ShapeDtypeStruct((NC, NPAD, width), jnp.float32),
        mesh=_mesh,
        scratch_types=[
            pltpu.VMEM((NB, BL), jnp.int32),
            pltpu.VMEM((NB, BL), jnp.int32),
            pltpu.VMEM((BL, width), jnp.float32),
            pltpu.VMEM_SHARED((NPAD, width), jnp.float32),
            pltpu.SemaphoreType.DMA,
        ],
    )
    def _sc_conv(src_hbm, dst_hbm, g_hbm, zeros_hbm, out_hbm,
                 isrc_v, idst_v, rows_v, acc_sh, sem):
        cid = lax.axis_index("c")
        sid = lax.axis_index("s")
        t = cid * NS + sid
        pltpu.sync_copy(src_hbm.at[t], isrc_v)
        pltpu.sync_copy(dst_hbm.at[t], idst_v)
        r0 = sid * RPT
        pltpu.sync_copy(zeros_hbm.at[pl.ds(r0, RPT)], acc_sh.at[pl.ds(r0, RPT)])
        plsc.subcore_barrier()

        def body(j, carry):
            pltpu.async_copy(g_hbm.at[isrc_v.at[j]], rows_v, sem).wait()
            pltpu.sync_copy(rows_v, acc_sh.at[idst_v.at[j]], add=True)
            return carry

        lax.fori_loop(0, NB, body, 0)
        plsc.subcore_barrier()
        pltpu.sync_copy(acc_sh.at[pl.ds(r0, RPT)],
                        out_hbm.at[cid, pl.ds(r0, RPT)])

    return _sc_conv


_sc_conv_d = _make_sc_conv(D)
_sc_conv_c = _make_sc_conv(CP)


def _dinv_from_hist(hist):
    deg = hist[0, :N, 0:1] + hist[1, :N, 0:1] + 1.0
    return lax.rsqrt(deg)


def _tc_a_body(x_ref, w1_ref, hist_ref, g1_ref):
    dinv = _dinv_from_hist(hist_ref[...])
    g1_ref[...] = dinv * jnp.dot(x_ref[...], w1_ref[...],
                                 preferred_element_type=jnp.float32)


def _tc_b_body(hist_ref, s1_ref, g1_ref, b1_ref, gamma_ref, beta_ref,
               w2_ref, g2_ref):
    dinv = _dinv_from_hist(hist_ref[...])
    s1 = s1_ref[...]
    h = dinv * (s1[0, :N, :] + s1[1, :N, :] + g1_ref[...]) + b1_ref[...]
    mean = jnp.mean(h, axis=0, keepdims=True)
    var = jnp.mean((h - mean) ** 2, axis=0, keepdims=True)
    hn = (h - mean) * lax.rsqrt(var + 1e-5) * gamma_ref[...] + beta_ref[...]
    hr = jnp.maximum(hn, 0.0)
    g2_ref[...] = dinv * jnp.dot(hr, w2_ref[...],
                                 preferred_element_type=jnp.float32)


def _tc_c_body(hist_ref, s2_ref, g2_ref, b2_ref, out_ref):
    dinv = _dinv_from_hist(hist_ref[...])
    s2 = s2_ref[...]
    out_ref[...] = dinv * (s2[0, :N, :] + s2[1, :N, :] + g2_ref[...]) \
        + b2_ref[...]


def kernel(x, edge_index, W1, b1, gamma, beta, W2, b2):
    src = edge_index[0]
    dst = edge_index[1]
    src_p = jnp.concatenate(
        [src, jnp.zeros((PADE,), jnp.int32)]).reshape(NT, NB, BL)
    dst_p = jnp.concatenate(
        [dst, jnp.full((PADE,), N, jnp.int32)]).reshape(NT, NB, BL)

    zerosD = jnp.zeros((NPAD, D), jnp.float32)
    onesD = jnp.ones((BL, D), jnp.float32)

    hist = _sc_hist(dst_p, zerosD, onesD)

    g1 = pl.pallas_call(
        _tc_a_body,
        out_shape=jax.ShapeDtypeStruct((N, D), jnp.float32),
    )(x, W1, hist)

    s1 = _sc_conv_d(src_p, dst_p, g1, zerosD)

    W2p = jnp.pad(W2, ((0, 0), (0, CP - C)))
    g2 = pl.pallas_call(
        _tc_b_body,
        out_shape=jax.ShapeDtypeStruct((N, CP), jnp.float32),
    )(hist, s1, g1, b1.reshape(1, D), gamma.reshape(1, D),
      beta.reshape(1, D), W2p)

    s2 = _sc_conv_c(src_p, dst_p, g2, zerosD)

    b2p = jnp.pad(b2, (0, CP - C)).reshape(1, CP)
    out = pl.pallas_call(
        _tc_c_body,
        out_shape=jax.ShapeDtypeStruct((N, CP), jnp.float32),
    )(hist, s2, g2, b2p)

    return out[:, :C]


# speedup vs baseline: 2.3951x; 2.3951x over previous
import functools

import jax
import jax.numpy as jnp
from jax import lax
from jax.experimental import pallas as pl
from jax.experimental.pallas import tpu as pltpu
from jax.experimental.pallas import tpu_sc as plsc

N = 10000
E = 320000
D = 128
C = 40
CP = 128

NC = 2
NS = 16
NT = NC * NS
BL = 128
NB = 79
EPT = NB * BL
PADE = NT * EPT - E
NPAD = 10112
RPT = NPAD // NS

_mesh = plsc.VectorSubcoreMesh(
    core_axis_name="c", subcore_axis_name="s", num_cores=NC, num_subcores=NS)


@functools.partial(
    pl.kernel,
    out_type=jax.ShapeDtypeStruct((NC, NPAD, D), jnp.float32),
    mesh=_mesh,
    scratch_types=[
        pltpu.VMEM((NB, BL), jnp.int32),
        pltpu.VMEM((BL, D), jnp.float32),
        pltpu.VMEM_SHARED((NPAD, D), jnp.float32),
    ],
)
def _sc_hist(dst_hbm, zeros_hbm, ones_hbm, out_hbm, idx_v, ones_v, acc_sh):
    cid = lax.axis_index("c")
    sid = lax.axis_index("s")
    t = cid * NS + sid
    pltpu.sync_copy(dst_hbm.at[t], idx_v)
    pltpu.sync_copy(ones_hbm, ones_v)
    r0 = sid * RPT
    pltpu.sync_copy(zeros_hbm.at[pl.ds(r0, RPT)], acc_sh.at[pl.ds(r0, RPT)])
    plsc.subcore_barrier()

    def body(j, carry):
        pltpu.sync_copy(ones_v, acc_sh.at[idx_v.at[j]], add=True)
        return carry

    lax.fori_loop(0, NB, body, 0)
    plsc.subcore_barrier()
    pltpu.sync_copy(acc_sh.at[pl.ds(r0, RPT)], out_hbm.at[cid, pl.ds(r0, RPT)])


def _make_sc_conv(width):
    @functools.partial(
        pl.kernel,
        out_type=jax.ShapeDtypeStruct((NC, NPAD, width), jnp.float32),
        mesh=_mesh,
        scratch_types=[
            pltpu.VMEM((NB, BL), jnp.int32),
            pltpu.VMEM((NB, BL), jnp.int32),
            pltpu.VMEM((BL, width), jnp.float32),
            pltpu.VMEM_SHARED((NPAD, width), jnp.float32),
            pltpu.SemaphoreType.DMA,
        ],
    )
    def _sc_conv(src_hbm, dst_hbm, g_hbm, zeros_hbm, out_hbm,
                 isrc_v, idst_v, rows_v, acc_sh, sem):
        cid = lax.axis_index("c")
        sid = lax.axis_index("s")
        t = cid * NS + sid
        pltpu.sync_copy(src_hbm.at[t], isrc_v)
        pltpu.sync_copy(dst_hbm.at[t], idst_v)
        r0 = sid * RPT
        pltpu.sync_copy(zeros_hbm.at[pl.ds(r0, RPT)], acc_sh.at[pl.ds(r0, RPT)])
        plsc.subcore_barrier()

        def body(j, carry):
            pltpu.async_copy(g_hbm.at[isrc_v.at[j]], rows_v, sem).wait()
            pltpu.sync_copy(rows_v, acc_sh.at[idst_v.at[j]], add=True)
            return carry

        lax.fori_loop(0, NB, body, 0)
        plsc.subcore_barrier()
        pltpu.sync_copy(acc_sh.at[pl.ds(r0, RPT)],
                        out_hbm.at[cid, pl.ds(r0, RPT)])

    return _sc_conv


_sc_conv_d = _make_sc_conv(D)
_sc_conv_c = _make_sc_conv(CP)


def _dinv_from_hist(hist):
    deg = hist[0, :N, 0:1] + hist[1, :N, 0:1] + 1.0
    return lax.rsqrt(deg)


def _tc_a_body(x_ref, w1_ref, hist_ref, g1_ref):
    dinv = _dinv_from_hist(hist_ref[...])
    g1_ref[...] = dinv * jnp.dot(x_ref[...], w1_ref[...],
                                 preferred_element_type=jnp.float32)


def _tc_b_body(hist_ref, s1_ref, g1_ref, b1_ref, gamma_ref, beta_ref,
               w2_ref, g2_ref):
    dinv = _dinv_from_hist(hist_ref[...])
    s1 = s1_ref[...]
    h = dinv * (s1[0, :N, :] + s1[1, :N, :] + g1_ref[...]) + b1_ref[...]
    mean = jnp.mean(h, axis=0, keepdims=True)
    var = jnp.mean((h - mean) ** 2, axis=0, keepdims=True)
    hn = (h - mean) * lax.rsqrt(var + 1e-5) * gamma_ref[...] + beta_ref[...]
    hr = jnp.maximum(hn, 0.0)
    g2_ref[...] = dinv * jnp.dot(hr, w2_ref[...],
                                 preferred_element_type=jnp.float32)


def _tc_c_body(hist_ref, s2_ref, g2_ref, b2_ref, out_ref):
    dinv = _dinv_from_hist(hist_ref[...])
    s2 = s2_ref[...]
    out_ref[...] = dinv * (s2[0, :N, :] + s2[1, :N, :] + g2_ref[...]) \
        + b2_ref[...]


def kernel(x, edge_index, W1, b1, gamma, beta, W2, b2):
    src = edge_index[0]
    dst = edge_index[1]
    pad_iota = jnp.arange(PADE, dtype=jnp.int32)
    src_p = jnp.concatenate(
        [src, pad_iota % N]).reshape(NT, NB, BL)
    dst_p = jnp.concatenate(
        [dst, N + pad_iota % (NPAD - N)]).reshape(NT, NB, BL)

    zerosD = jnp.zeros((NPAD, D), jnp.float32)
    onesD = jnp.ones((BL, D), jnp.float32)

    hist = _sc_hist(dst_p, zerosD, onesD)

    g1 = pl.pallas_call(
        _tc_a_body,
        out_shape=jax.ShapeDtypeStruct((N, D), jnp.float32),
    )(x, W1, hist)

    s1 = _sc_conv_d(src_p, dst_p, g1, zerosD)

    W2p = jnp.pad(W2, ((0, 0), (0, CP - C)))
    g2 = pl.pallas_call(
        _tc_b_body,
        out_shape=jax.ShapeDtypeStruct((N, CP), jnp.float32),
    )(hist, s1, g1, b1.reshape(1, D), gamma.reshape(1, D),
      beta.reshape(1, D), W2p)

    s2 = _sc_conv_c(src_p, dst_p, g2, zerosD)

    b2p = jnp.pad(b2, (0, CP - C)).reshape(1, CP)
    out = pl.pallas_call(
        _tc_c_body,
        out_shape=jax.ShapeDtypeStruct((N, CP), jnp.float32),
    )(hist, s2, g2, b2p)

    return out[:, :C]


# speedup vs baseline: 2.8176x; 1.1764x over previous
import functools

import jax
import jax.numpy as jnp
from jax import lax
from jax.experimental import pallas as pl
from jax.experimental.pallas import tpu as pltpu
from jax.experimental.pallas import tpu_sc as plsc

N = 10000
E = 320000
D = 128
C = 40
CP = 128

NC = 2
NS = 16
NT = NC * NS
BL = 128
NB = 80
NBC = 16
EPT = NB * BL
PADE = NT * EPT - E
NPAD = 10112
RPT = NPAD // NS

_mesh = plsc.VectorSubcoreMesh(
    core_axis_name="c", subcore_axis_name="s", num_cores=NC, num_subcores=NS)


@functools.partial(
    pl.kernel,
    out_type=jax.ShapeDtypeStruct((NC, NPAD, D), jnp.float32),
    mesh=_mesh,
    scratch_types=[
        pltpu.VMEM((NB, BL), jnp.int32),
        pltpu.VMEM((BL, D), jnp.float32),
        pltpu.VMEM_SHARED((NPAD, D), jnp.float32),
    ],
)
def _sc_hist(dst_hbm, zeros_hbm, ones_hbm, out_hbm, idx_v, ones_v, acc_sh):
    cid = lax.axis_index("c")
    sid = lax.axis_index("s")
    t = cid * NS + sid
    pltpu.sync_copy(dst_hbm.at[t], idx_v)
    pltpu.sync_copy(ones_hbm, ones_v)
    r0 = sid * RPT
    pltpu.sync_copy(zeros_hbm.at[pl.ds(r0, RPT)], acc_sh.at[pl.ds(r0, RPT)])
    plsc.subcore_barrier()

    def body(j, carry):
        pltpu.sync_copy(ones_v, acc_sh.at[idx_v.at[j]], add=True)
        return carry

    lax.fori_loop(0, NB, body, 0)
    plsc.subcore_barrier()
    pltpu.sync_copy(acc_sh.at[pl.ds(r0, RPT)], out_hbm.at[cid, pl.ds(r0, RPT)])


def _make_sc_conv(width):
    @functools.partial(
        pl.kernel,
        out_type=jax.ShapeDtypeStruct((NC, NPAD, width), jnp.float32),
        mesh=_mesh,
        scratch_types=[
            pltpu.VMEM((NBC, BL), jnp.int32),
            pltpu.VMEM((NBC, BL), jnp.int32),
            pltpu.VMEM((BL, width), jnp.float32),
            pltpu.VMEM((BL, width), jnp.float32),
            pltpu.VMEM_SHARED((NPAD, width), jnp.float32),
            pltpu.SemaphoreType.DMA,
            pltpu.SemaphoreType.DMA,
        ],
    )
    def _sc_conv(src_hbm, dst_hbm, g_hbm, zeros_hbm, out_hbm,
                 isrc_v, idst_v, rows0_v, rows1_v, acc_sh, sem0, sem1):
        cid = lax.axis_index("c")
        sid = lax.axis_index("s")
        t = cid * NS + sid
        r0 = sid * RPT
        pltpu.sync_copy(zeros_hbm.at[pl.ds(r0, RPT)], acc_sh.at[pl.ds(r0, RPT)])
        plsc.subcore_barrier()

        def chunk(c, carry):
            pltpu.sync_copy(src_hbm.at[t, pl.ds(c * NBC, NBC)], isrc_v)
            pltpu.sync_copy(dst_hbm.at[t, pl.ds(c * NBC, NBC)], idst_v)
            pltpu.async_copy(g_hbm.at[isrc_v.at[0]], rows0_v, sem0)

            def body(i, carry2):
                j = 2 * i
                pltpu.make_async_copy(g_hbm.at[isrc_v.at[j]], rows0_v,
                                      sem0).wait()
                pltpu.async_copy(g_hbm.at[isrc_v.at[j + 1]], rows1_v, sem1)
                pltpu.sync_copy(rows0_v, acc_sh.at[idst_v.at[j]], add=True)
                pltpu.make_async_copy(g_hbm.at[isrc_v.at[j + 1]], rows1_v,
                                      sem1).wait()

                @pl.when(i + 1 < NBC // 2)
                def _():
                    pltpu.async_copy(g_hbm.at[isrc_v.at[j + 2]], rows0_v,
                                     sem0)

                pltpu.sync_copy(rows1_v, acc_sh.at[idst_v.at[j + 1]],
                                add=True)
                return carry2

            lax.fori_loop(0, NBC // 2, body, 0)
            return carry

        lax.fori_loop(0, NB // NBC, chunk, 0)
        plsc.subcore_barrier()
        pltpu.sync_copy(acc_sh.at[pl.ds(r0, RPT)],
                        out_hbm.at[cid, pl.ds(r0, RPT)])

    return _sc_conv


_sc_conv_d = _make_sc_conv(D)
_sc_conv_c = _make_sc_conv(CP)


def _dinv_from_hist(hist):
    deg = hist[0, :N, 0:1] + hist[1, :N, 0:1] + 1.0
    return lax.rsqrt(deg)


def _tc_a_body(x_ref, w1_ref, hist_ref, g1_ref):
    dinv = _dinv_from_hist(hist_ref[...])
    g1_ref[...] = dinv * jnp.dot(x_ref[...], w1_ref[...],
                                 preferred_element_type=jnp.float32)


def _tc_b_body(hist_ref, s1_ref, g1_ref, b1_ref, gamma_ref, beta_ref,
               w2_ref, g2_ref):
    dinv = _dinv_from_hist(hist_ref[...])
    s1 = s1_ref[...]
    h = dinv * (s1[0, :N, :] + s1[1, :N, :] + g1_ref[...]) + b1_ref[...]
    mean = jnp.mean(h, axis=0, keepdims=True)
    var = jnp.mean((h - mean) ** 2, axis=0, keepdims=True)
    hn = (h - mean) * lax.rsqrt(var + 1e-5) * gamma_ref[...] + beta_ref[...]
    hr = jnp.maximum(hn, 0.0)
    g2_ref[...] = dinv * jnp.dot(hr, w2_ref[...],
                                 preferred_element_type=jnp.float32)


def _tc_c_body(hist_ref, s2_ref, g2_ref, b2_ref, out_ref):
    dinv = _dinv_from_hist(hist_ref[...])
    s2 = s2_ref[...]
    out_ref[...] = dinv * (s2[0, :N, :] + s2[1, :N, :] + g2_ref[...]) \
        + b2_ref[...]


def kernel(x, edge_index, W1, b1, gamma, beta, W2, b2):
    src = edge_index[0]
    dst = edge_index[1]
    pad_iota = jnp.arange(PADE, dtype=jnp.int32)
    src_p = jnp.concatenate(
        [src, pad_iota % N]).reshape(NT, NB, BL)
    dst_p = jnp.concatenate(
        [dst, N + pad_iota % (NPAD - N)]).reshape(NT, NB, BL)

    zerosD = jnp.zeros((NPAD, D), jnp.float32)
    onesD = jnp.ones((BL, D), jnp.float32)

    hist = _sc_hist(dst_p, zerosD, onesD)

    g1 = pl.pallas_call(
        _tc_a_body,
        out_shape=jax.ShapeDtypeStruct((N, D), jnp.float32),
    )(x, W1, hist)

    s1 = _sc_conv_d(src_p, dst_p, g1, zerosD)

    W2p = jnp.pad(W2, ((0, 0), (0, CP - C)))
    g2 = pl.pallas_call(
        _tc_b_body,
        out_shape=jax.ShapeDtypeStruct((N, CP), jnp.float32),
    )(hist, s1, g1, b1.reshape(1, D), gamma.reshape(1, D),
      beta.reshape(1, D), W2p)

    s2 = _sc_conv_c(src_p, dst_p, g2, zerosD)

    b2p = jnp.pad(b2, (0, CP - C)).reshape(1, CP)
    out = pl.pallas_call(
        _tc_c_body,
        out_shape=jax.ShapeDtypeStruct((N, CP), jnp.float32),
    )(hist, s2, g2, b2p)

    return out[:, :C]
